# baseline (speedup 1.0000x reference)
import functools

import jax
import jax.numpy as jnp
from jax import lax
from jax.experimental import pallas as pl
from jax.experimental.pallas import tpu as pltpu
from jax.experimental.pallas import tpu_sc as plsc

POOLED = 7
SCALE = 0.0625
MIN_SIZE = 16.0
NEG = -1e30
WIN = 16
NUM_WORKERS = 32
CGRP = 16


def _anchor_body(blk_ref, rois_ref, scal_ref, cx1_ref, cy1_ref, cx2_ref,
                 cy2_ref, r0_ref, r1_ref, *, H, W):
    f32 = jnp.float32
    i32 = jnp.int32
    blk = blk_ref[...]
    rois = rois_ref[...]
    N = rois.shape[0]
    hh = scal_ref[0, 0]
    hw = scal_ref[0, 1]
    x1 = blk[:, 1:2]
    y1 = blk[:, 2:3]
    x2 = blk[:, 3:4]
    y2 = blk[:, 4:5]
    w = x2 - x1
    h = y2 - y1
    ci = lax.broadcasted_iota(i32, (1, 8), 1)
    ci = jnp.where(ci >= 4, ci + 1, ci)
    ix = (ci % 3).astype(f32)
    iy = (ci // 3).astype(f32)
    sx = x1 - w + w * ix + w / 2.0
    sy = y1 - h + h * iy + h / 2.0
    gx1 = sx - w / 4.0
    gy1 = sy - h / 4.0
    gx2 = sx + w / 4.0
    gy2 = sy + h / 4.0
    ww = gx2 - gx1 + 1.0
    hhh = gy2 - gy1 + 1.0
    keep = ((gx1 < 0) | (gy1 < 0) | (gx2 >= hw) | (gy2 >= hh)
            | (ww < MIN_SIZE) | (hhh < MIN_SIZE))
    gx1 = jnp.where(keep, x1, gx1)
    gy1 = jnp.where(keep, y1, gy1)
    gx2 = jnp.where(keep, x2, gx2)
    gy2 = jnp.where(keep, y2, gy2)

    ax1 = rois[:, 1].reshape(1, 1, N)
    ay1 = rois[:, 2].reshape(1, 1, N)
    ax2 = rois[:, 3].reshape(1, 1, N)
    ay2 = rois[:, 4].reshape(1, 1, N)
    an_w = ax2 - ax1 + 1.0
    an_h = ay2 - ay1 + 1.0
    an_area = an_w * an_h
    g1 = gx1[:, :, None]
    g2 = gy1[:, :, None]
    g3 = gx2[:, :, None]
    g4 = gy2[:, :, None]
    gt_w = gx2 - gx1 + 1.0
    gt_h = gy2 - gy1 + 1.0
    gt_area = (gt_w * gt_h)[:, :, None]
    iw = jnp.maximum(jnp.minimum(ax2, g3) - jnp.maximum(ax1, g1) + 1.0, 0.0)
    ih = jnp.maximum(jnp.minimum(ay2, g4) - jnp.maximum(ay1, g2) + 1.0, 0.0)
    inter = iw * ih
    ua = an_area + gt_area - inter
    ov = inter / ua
    gt_zero = ((gt_w == 1.0) & (gt_h == 1.0)).astype(f32)[:, :, None]
    an_zero = (an_w == 1.0) & (an_h == 1.0)
    ov = jnp.where(gt_zero == 1.0, 0.0, ov)
    ov = jnp.where(an_zero, -1.0, ov)

    BN = blk.shape[0]
    gt_max = jnp.max(ov, axis=2)
    niota = lax.broadcasted_iota(i32, (BN, 8, N), 2)
    amax1 = jnp.min(jnp.where(ov == gt_max[:, :, None], niota, 2 * N), axis=2)
    onehot = (niota == amax1[:, :, None]).astype(f32)
    selx1 = jnp.sum(onehot * ax1, axis=2)
    sely1 = jnp.sum(onehot * ay1, axis=2)
    selx2 = jnp.sum(onehot * ax2, axis=2)
    sely2 = jnp.sum(onehot * ay2, axis=2)

    gt_max_adj = jnp.where(gt_max == 0.0, 1e-5, gt_max)
    labels = jnp.where(gt_max_adj >= 0.3, 1.0, 0.0)
    wcell = gx2 - gx1
    hcell = gy2 - gy1
    maxc = jnp.maximum(wcell, hcell)
    minc = jnp.minimum(wcell, hcell)
    pos = labels == 1.0
    wsel = selx2 - selx1
    hsel = sely2 - sely1
    width = jnp.where(pos, wsel, 0.0)
    height = jnp.where(pos, hsel, 0.0)
    mx = jnp.maximum(width, height)
    mn = jnp.minimum(width, height)
    labels = jnp.where(mx >= maxc, 0.0, labels)
    labels = jnp.where(mn < (1.0 / 3.0) * minc, 0.0, labels)
    pos2 = labels == 1.0
    gx1 = jnp.where(pos2, selx1, gx1)
    gy1 = jnp.where(pos2, sely1, gy1)
    gx2 = jnp.where(pos2, selx2, gx2)
    gy2 = jnp.where(pos2, sely2, gy2)

    cx1 = jnp.concatenate([x1, gx1], axis=1)
    cy1 = jnp.concatenate([y1, gy1], axis=1)
    cx2 = jnp.concatenate([x2, gx2], axis=1)
    cy2 = jnp.concatenate([y2, gy2], axis=1)
    cx1_ref[...] = cx1
    cy1_ref[...] = cy1
    cx2_ref[...] = cx2
    cy2_ref[...] = cy2

    x1c = jnp.round(cx1 * SCALE).astype(i32)
    y1c = jnp.round(cy1 * SCALE).astype(i32)
    x2c = jnp.round(cx2 * SCALE).astype(i32)
    y2c = jnp.round(cy2 * SCALE).astype(i32)
    roi_w = jnp.maximum(x2c - x1c + 1, 1).astype(f32)
    roi_h = jnp.maximum(y2c - y1c + 1, 1).astype(f32)
    bwf = (roi_w / float(POOLED))[:, :, None]
    bhf = (roi_h / float(POOLED))[:, :, None]
    ib = lax.broadcasted_iota(i32, (BN, 9, POOLED), 2).astype(f32)
    y1e = y1c[:, :, None]
    x1e = x1c[:, :, None]
    hs = jnp.clip(jnp.floor(ib * bhf).astype(i32) + y1e, 0, H)
    he = jnp.clip(jnp.ceil((ib + 1.0) * bhf).astype(i32) + y1e, 0, H)
    ws = jnp.clip(jnp.floor(ib * bwf).astype(i32) + x1e, 0, W)
    we = jnp.clip(jnp.ceil((ib + 1.0) * bwf).astype(i32) + x1e, 0, W)
    hc = jnp.minimum(y1c, H - WIN)[:, :, None]
    wc = jnp.minimum(x1c, W - WIN)[:, :, None]
    zz = jnp.zeros((BN, 9, 1), i32)
    r0_ref[...] = jnp.concatenate([hs, he, hc, zz], axis=2)
    r1_ref[...] = jnp.concatenate([ws, we, wc, zz], axis=2)


def _make_sc_pool(n_rois, H, W, C):
    per_w = n_rois // NUM_WORKERS
    mesh = plsc.VectorSubcoreMesh(core_axis_name="c", subcore_axis_name="s")
    nbins = POOLED * POOLED

    HGRP = CGRP // 2

    @functools.partial(
        pl.kernel,
        out_type=jax.ShapeDtypeStruct((nbins, n_rois, CGRP, 16), jnp.float32),
        mesh=mesh,
        compiler_params=pltpu.CompilerParams(needs_layout_passes=False,
                                             use_tc_tiling_on_sc=False),
        scratch_types=[
            pltpu.VMEM((WIN, WIN, HGRP, 16), jnp.float32),
            pltpu.VMEM((WIN, WIN, HGRP, 16), jnp.float32),
            pltpu.VMEM((per_w * 16 + 16,), jnp.int32),
            pltpu.VMEM((per_w * 16 + 16,), jnp.int32),
            pltpu.VMEM((nbins, CGRP, 16), jnp.float32),
            pltpu.SemaphoreType.DMA,
            pltpu.SemaphoreType.DMA,
        ],
    )
    def sc_pool(feat_hbm, r0_hbm, r1_hbm, out_hbm, bufa, bufb, b0, b1, obuf,
                sema, semb):
        wid = lax.axis_index("s") * 2 + lax.axis_index("c")
        base = wid * per_w
        pltpu.sync_copy(r0_hbm.at[pl.ds(base * 16, per_w * 16)],
                        b0.at[pl.ds(0, per_w * 16)])
        pltpu.sync_copy(r1_hbm.at[pl.ds(base * 16, per_w * 16)],
                        b1.at[pl.ds(0, per_w * 16)])

        def _at(ref, off):
            return ref[pl.ds(off, 16)][0]

        def _half(hc, wc, h):
            return feat_hbm.at[pl.ds(hc, WIN), pl.ds(wc, WIN),
                               pl.ds(h * HGRP, HGRP)]

        hc0 = _at(b0, 14)
        wc0 = _at(b1, 14)
        pltpu.async_copy(_half(hc0, wc0, 0), bufa, sema)

        def roi_body(rl, carry):
            rb = rl * 16
            row0 = b0[pl.ds(rb, 16)]
            row1 = b1[pl.ds(rb, 16)]
            hc = row0[14]
            wc = row1[14]
            rn = jnp.minimum(rl + 1, per_w - 1) * 16
            hcn = _at(b0, rn + 14)
            wcn = _at(b1, rn + 14)
            hs_l = [row0[i] - hc for i in range(POOLED)]
            he_l = [row0[i + POOLED] - hc for i in range(POOLED)]
            ws_l = [row1[j] - wc for j in range(POOLED)]
            we_l = [row1[j + POOLED] - wc for j in range(POOLED)]

            for h in range(2):
                buf, sem = (bufa, sema) if h == 0 else (bufb, semb)
                pltpu.make_async_copy(_half(hc, wc, h), buf, sem).wait()
                if h == 0:
                    pltpu.async_copy(_half(hc, wc, 1), bufb, semb)
                else:
                    pltpu.async_copy(_half(hcn, wcn, 0), bufa, sema)

                for i in range(POOLED):
                    hs, he = hs_l[i], he_l[i]
                    for j in range(POOLED):
                        ws, we = ws_l[j], we_l[j]

                        def row_red(r, acc):
                            def col_red(t, acc2):
                                return tuple(
                                    jnp.maximum(acc2[g], buf[r, t, g, :])
                                    for g in range(HGRP))
                            return lax.fori_loop(ws, we, col_red, acc)

                        acc0 = tuple(jnp.full((16,), NEG, jnp.float32)
                                     for _ in range(HGRP))
                        acc = lax.fori_loop(hs, he, row_red, acc0)
                        empty = (we <= ws) | (he <= hs)
                        k = i * POOLED + j
                        for g in range(HGRP):
                            obuf[k, h * HGRP + g, :] = jnp.where(
                                empty, 0.0, acc[g])

            pltpu.sync_copy(obuf, out_hbm.at[:, base + rl])
            return carry

        lax.fori_loop(0, per_w, roi_body, 0)
        pltpu.make_async_copy(_half(0, 0, 0), bufa, sema).wait()

    return sc_pool


def kernel(features, rois, hh, hw):
    B, C, H, W = features.shape
    N = rois.shape[0]
    f32 = jnp.float32
    scal = jnp.stack([jnp.asarray(hh, f32),
                      jnp.asarray(hw, f32)]).reshape(1, 2)
    body = functools.partial(_anchor_body, H=H, W=W)
    BN = 16
    grid = N // BN
    cx1, cy1, cx2, cy2, r0, r1 = pl.pallas_call(
        body,
        grid=(grid,),
        out_shape=[
            jax.ShapeDtypeStruct((N, 9), f32),
            jax.ShapeDtypeStruct((N, 9), f32),
            jax.ShapeDtypeStruct((N, 9), f32),
            jax.ShapeDtypeStruct((N, 9), f32),
            jax.ShapeDtypeStruct((N, 9, 16), jnp.int32),
            jax.ShapeDtypeStruct((N, 9, 16), jnp.int32),
        ],
        in_specs=[
            pl.BlockSpec((BN, 5), lambda i: (i, 0)),
            pl.BlockSpec((N, 5), lambda i: (0, 0)),
            pl.BlockSpec((1, 2), lambda i: (0, 0),
                         memory_space=pltpu.SMEM),
        ],
        out_specs=[
            pl.BlockSpec((BN, 9), lambda i: (i, 0)),
            pl.BlockSpec((BN, 9), lambda i: (i, 0)),
            pl.BlockSpec((BN, 9), lambda i: (i, 0)),
            pl.BlockSpec((BN, 9), lambda i: (i, 0)),
            pl.BlockSpec((BN, 9, 16), lambda i: (i, 0, 0)),
            pl.BlockSpec((BN, 9, 16), lambda i: (i, 0, 0)),
        ],
    )(rois, rois, scal)

    catb = jnp.concatenate([rois[:, :1], jnp.zeros((N, 8), f32)], axis=1)
    cat_rois = jnp.stack([catb, cx1, cy1, cx2, cy2], axis=-1).reshape(-1, 5)

    featT = jnp.transpose(features[0], (1, 2, 0)).reshape(H, W, CGRP, 16)
    n_rois = 9 * N
    sc_pool = _make_sc_pool(n_rois, H, W, C)
    pooled = sc_pool(featT, r0.reshape(n_rois * 16), r1.reshape(n_rois * 16))
    pool_feat = pooled.reshape(POOLED, POOLED, N, 9 * C).transpose(2, 3, 0, 1)
    return (pool_feat, cat_rois)


# speedup vs baseline: 1.3972x; 1.3972x over previous
import functools

import jax
import jax.numpy as jnp
from jax import lax
from jax.experimental import pallas as pl
from jax.experimental.pallas import tpu as pltpu
from jax.experimental.pallas import tpu_sc as plsc

POOLED = 7
SCALE = 0.0625
MIN_SIZE = 16.0
NEG = -1e30
WIN = 16
NUM_WORKERS = 32
CGRP = 16


def _anchor_body(blk_ref, rois_ref, scal_ref, cx1_ref, cy1_ref, cx2_ref,
                 cy2_ref, r0_ref, r1_ref, *, H, W):
    f32 = jnp.float32
    i32 = jnp.int32
    blk = blk_ref[...]
    rois = rois_ref[...]
    N = rois.shape[0]
    hh = scal_ref[0, 0]
    hw = scal_ref[0, 1]
    x1 = blk[:, 1:2]
    y1 = blk[:, 2:3]
    x2 = blk[:, 3:4]
    y2 = blk[:, 4:5]
    w = x2 - x1
    h = y2 - y1
    ci = lax.broadcasted_iota(i32, (1, 8), 1)
    ci = jnp.where(ci >= 4, ci + 1, ci)
    ix = (ci % 3).astype(f32)
    iy = (ci // 3).astype(f32)
    sx = x1 - w + w * ix + w / 2.0
    sy = y1 - h + h * iy + h / 2.0
    gx1 = sx - w / 4.0
    gy1 = sy - h / 4.0
    gx2 = sx + w / 4.0
    gy2 = sy + h / 4.0
    ww = gx2 - gx1 + 1.0
    hhh = gy2 - gy1 + 1.0
    keep = ((gx1 < 0) | (gy1 < 0) | (gx2 >= hw) | (gy2 >= hh)
            | (ww < MIN_SIZE) | (hhh < MIN_SIZE))
    gx1 = jnp.where(keep, x1, gx1)
    gy1 = jnp.where(keep, y1, gy1)
    gx2 = jnp.where(keep, x2, gx2)
    gy2 = jnp.where(keep, y2, gy2)

    ax1 = rois[:, 1].reshape(1, 1, N)
    ay1 = rois[:, 2].reshape(1, 1, N)
    ax2 = rois[:, 3].reshape(1, 1, N)
    ay2 = rois[:, 4].reshape(1, 1, N)
    an_w = ax2 - ax1 + 1.0
    an_h = ay2 - ay1 + 1.0
    an_area = an_w * an_h
    g1 = gx1[:, :, None]
    g2 = gy1[:, :, None]
    g3 = gx2[:, :, None]
    g4 = gy2[:, :, None]
    gt_w = gx2 - gx1 + 1.0
    gt_h = gy2 - gy1 + 1.0
    gt_area = (gt_w * gt_h)[:, :, None]
    iw = jnp.maximum(jnp.minimum(ax2, g3) - jnp.maximum(ax1, g1) + 1.0, 0.0)
    ih = jnp.maximum(jnp.minimum(ay2, g4) - jnp.maximum(ay1, g2) + 1.0, 0.0)
    inter = iw * ih
    ua = an_area + gt_area - inter
    ov = inter / ua
    gt_zero = ((gt_w == 1.0) & (gt_h == 1.0)).astype(f32)[:, :, None]
    an_zero = (an_w == 1.0) & (an_h == 1.0)
    ov = jnp.where(gt_zero == 1.0, 0.0, ov)
    ov = jnp.where(an_zero, -1.0, ov)

    BN = blk.shape[0]
    gt_max = jnp.max(ov, axis=2)
    niota = lax.broadcasted_iota(i32, (BN, 8, N), 2)
    amax1 = jnp.min(jnp.where(ov == gt_max[:, :, None], niota, 2 * N), axis=2)
    onehot = (niota == amax1[:, :, None]).astype(f32)
    selx1 = jnp.sum(onehot * ax1, axis=2)
    sely1 = jnp.sum(onehot * ay1, axis=2)
    selx2 = jnp.sum(onehot * ax2, axis=2)
    sely2 = jnp.sum(onehot * ay2, axis=2)

    gt_max_adj = jnp.where(gt_max == 0.0, 1e-5, gt_max)
    labels = jnp.where(gt_max_adj >= 0.3, 1.0, 0.0)
    wcell = gx2 - gx1
    hcell = gy2 - gy1
    maxc = jnp.maximum(wcell, hcell)
    minc = jnp.minimum(wcell, hcell)
    pos = labels == 1.0
    wsel = selx2 - selx1
    hsel = sely2 - sely1
    width = jnp.where(pos, wsel, 0.0)
    height = jnp.where(pos, hsel, 0.0)
    mx = jnp.maximum(width, height)
    mn = jnp.minimum(width, height)
    labels = jnp.where(mx >= maxc, 0.0, labels)
    labels = jnp.where(mn < (1.0 / 3.0) * minc, 0.0, labels)
    pos2 = labels == 1.0
    gx1 = jnp.where(pos2, selx1, gx1)
    gy1 = jnp.where(pos2, sely1, gy1)
    gx2 = jnp.where(pos2, selx2, gx2)
    gy2 = jnp.where(pos2, sely2, gy2)

    cx1 = jnp.concatenate([x1, gx1], axis=1)
    cy1 = jnp.concatenate([y1, gy1], axis=1)
    cx2 = jnp.concatenate([x2, gx2], axis=1)
    cy2 = jnp.concatenate([y2, gy2], axis=1)
    cx1_ref[...] = cx1
    cy1_ref[...] = cy1
    cx2_ref[...] = cx2
    cy2_ref[...] = cy2

    x1c = jnp.round(cx1 * SCALE).astype(i32)
    y1c = jnp.round(cy1 * SCALE).astype(i32)
    x2c = jnp.round(cx2 * SCALE).astype(i32)
    y2c = jnp.round(cy2 * SCALE).astype(i32)
    roi_w = jnp.maximum(x2c - x1c + 1, 1).astype(f32)
    roi_h = jnp.maximum(y2c - y1c + 1, 1).astype(f32)
    bwf = (roi_w / float(POOLED))[:, :, None]
    bhf = (roi_h / float(POOLED))[:, :, None]
    ib = lax.broadcasted_iota(i32, (BN, 9, POOLED), 2).astype(f32)
    y1e = y1c[:, :, None]
    x1e = x1c[:, :, None]
    hs = jnp.clip(jnp.floor(ib * bhf).astype(i32) + y1e, 0, H)
    he = jnp.clip(jnp.ceil((ib + 1.0) * bhf).astype(i32) + y1e, 0, H)
    ws = jnp.clip(jnp.floor(ib * bwf).astype(i32) + x1e, 0, W)
    we = jnp.clip(jnp.ceil((ib + 1.0) * bwf).astype(i32) + x1e, 0, W)
    hc = jnp.minimum(y1c, H - WIN)[:, :, None]
    wc = jnp.minimum(x1c, W - WIN)[:, :, None]
    zz = jnp.zeros((BN, 9, 1), i32)
    r0_ref[...] = jnp.concatenate([hs, he, hc, zz], axis=2)
    r1_ref[...] = jnp.concatenate([ws, we, wc, zz], axis=2)


def _make_sc_pool(n_rois, H, W, C):
    per_w = n_rois // NUM_WORKERS
    mesh = plsc.VectorSubcoreMesh(core_axis_name="c", subcore_axis_name="s")
    nbins = POOLED * POOLED

    HGRP = CGRP // 2

    @functools.partial(
        pl.kernel,
        out_type=jax.ShapeDtypeStruct((nbins, n_rois, CGRP, 16), jnp.float32),
        mesh=mesh,
        compiler_params=pltpu.CompilerParams(needs_layout_passes=False,
                                             use_tc_tiling_on_sc=False),
        scratch_types=[
            pltpu.VMEM((WIN, WIN, HGRP, 16), jnp.float32),
            pltpu.VMEM((WIN, WIN, HGRP, 16), jnp.float32),
            pltpu.VMEM((per_w * 16 + 16,), jnp.int32),
            pltpu.VMEM((per_w * 16 + 16,), jnp.int32),
            pltpu.VMEM((nbins, CGRP, 16), jnp.float32),
            pltpu.SemaphoreType.DMA,
            pltpu.SemaphoreType.DMA,
        ],
    )
    def sc_pool(feat_hbm, r0_hbm, r1_hbm, out_hbm, bufa, bufb, b0, b1, obuf,
                sema, semb):
        wid = lax.axis_index("s") * 2 + lax.axis_index("c")
        base = wid * per_w
        pltpu.sync_copy(r0_hbm.at[pl.ds(base * 16, per_w * 16)],
                        b0.at[pl.ds(0, per_w * 16)])
        pltpu.sync_copy(r1_hbm.at[pl.ds(base * 16, per_w * 16)],
                        b1.at[pl.ds(0, per_w * 16)])

        def _at(ref, off):
            return ref[pl.ds(off, 16)][0]

        def _half(hc, wc, h):
            return feat_hbm.at[pl.ds(hc, WIN), pl.ds(wc, WIN),
                               pl.ds(h * HGRP, HGRP)]

        hc0 = _at(b0, 14)
        wc0 = _at(b1, 14)
        pltpu.async_copy(_half(hc0, wc0, 0), bufa, sema)

        def roi_body(rl, carry):
            rb = rl * 16
            hc = _at(b0, rb + 14)
            wc = _at(b1, rb + 14)
            rn = jnp.minimum(rl + 1, per_w - 1) * 16
            hcn = _at(b0, rn + 14)
            wcn = _at(b1, rn + 14)

            for h in range(2):
                buf, sem = (bufa, sema) if h == 0 else (bufb, semb)
                pltpu.make_async_copy(_half(hc, wc, h), buf, sem).wait()
                if h == 0:
                    pltpu.async_copy(_half(hc, wc, 1), bufb, semb)
                else:
                    pltpu.async_copy(_half(hcn, wcn, 0), bufa, sema)

                def bin_i(i, c0):
                    hs = _at(b0, rb + i) - hc
                    he = _at(b0, rb + i + POOLED) - hc

                    def bin_j(j, c1):
                        ws = _at(b1, rb + j) - wc
                        we = _at(b1, rb + j + POOLED) - wc

                        def row_red(r, acc):
                            def col_red(t, acc2):
                                return tuple(
                                    jnp.maximum(acc2[g], buf[r, t, g, :])
                                    for g in range(HGRP))
                            return lax.fori_loop(ws, we, col_red, acc)

                        acc0 = tuple(jnp.full((16,), NEG, jnp.float32)
                                     for _ in range(HGRP))
                        acc = lax.fori_loop(hs, he, row_red, acc0)
                        empty = (we <= ws) | (he <= hs)
                        k = i * POOLED + j
                        for g in range(HGRP):
                            obuf[k, h * HGRP + g, :] = jnp.where(
                                empty, 0.0, acc[g])
                        return c1

                    return lax.fori_loop(0, POOLED, bin_j, c0)

                lax.fori_loop(0, POOLED, bin_i, 0)

            pltpu.sync_copy(obuf, out_hbm.at[:, base + rl])
            return carry

        lax.fori_loop(0, per_w, roi_body, 0)
        pltpu.make_async_copy(_half(0, 0, 0), bufa, sema).wait()

    return sc_pool


def kernel(features, rois, hh, hw):
    B, C, H, W = features.shape
    N = rois.shape[0]
    f32 = jnp.float32
    scal = jnp.stack([jnp.asarray(hh, f32),
                      jnp.asarray(hw, f32)]).reshape(1, 2)
    body = functools.partial(_anchor_body, H=H, W=W)
    BN = 16
    grid = N // BN
    cx1, cy1, cx2, cy2, r0, r1 = pl.pallas_call(
        body,
        grid=(grid,),
        out_shape=[
            jax.ShapeDtypeStruct((N, 9), f32),
            jax.ShapeDtypeStruct((N, 9), f32),
            jax.ShapeDtypeStruct((N, 9), f32),
            jax.ShapeDtypeStruct((N, 9), f32),
            jax.ShapeDtypeStruct((N, 9, 16), jnp.int32),
            jax.ShapeDtypeStruct((N, 9, 16), jnp.int32),
        ],
        in_specs=[
            pl.BlockSpec((BN, 5), lambda i: (i, 0)),
            pl.BlockSpec((N, 5), lambda i: (0, 0)),
            pl.BlockSpec((1, 2), lambda i: (0, 0),
                         memory_space=pltpu.SMEM),
        ],
        out_specs=[
            pl.BlockSpec((BN, 9), lambda i: (i, 0)),
            pl.BlockSpec((BN, 9), lambda i: (i, 0)),
            pl.BlockSpec((BN, 9), lambda i: (i, 0)),
            pl.BlockSpec((BN, 9), lambda i: (i, 0)),
            pl.BlockSpec((BN, 9, 16), lambda i: (i, 0, 0)),
            pl.BlockSpec((BN, 9, 16), lambda i: (i, 0, 0)),
        ],
    )(rois, rois, scal)

    catb = jnp.concatenate([rois[:, :1], jnp.zeros((N, 8), f32)], axis=1)
    cat_rois = jnp.stack([catb, cx1, cy1, cx2, cy2], axis=-1).reshape(-1, 5)

    featT = jnp.transpose(features[0], (1, 2, 0)).reshape(H, W, CGRP, 16)
    n_rois = 9 * N
    sc_pool = _make_sc_pool(n_rois, H, W, C)
    pooled = sc_pool(featT, r0.reshape(n_rois * 16), r1.reshape(n_rois * 16))
    pool_feat = pooled.reshape(POOLED, POOLED, N, 9 * C).transpose(2, 3, 0, 1)
    return (pool_feat, cat_rois)
